# baseline (speedup 1.0000x reference)
import jax
import jax.numpy as jnp
from jax import lax
from jax.experimental import pallas as pl
from jax.experimental.pallas import tpu as pltpu
from jax.experimental.pallas import tpu_sc as plsc

_D = 32
_NC = 2
_NS = 16
_NW = _NC * _NS
_C = 512


def kernel(x, table):
    b, s = x.shape
    n = b * s
    idx = x.reshape(n)
    b_per_w = n // _NW
    n_chunks = b_per_w // _C
    mesh = plsc.VectorSubcoreMesh(core_axis_name="c", subcore_axis_name="s")

    @pl.kernel(
        out_type=jax.ShapeDtypeStruct((n, 4 * _D), table.dtype),
        mesh=mesh,
        compiler_params=pltpu.CompilerParams(use_tc_tiling_on_sc=False),
        scratch_types=[
            pltpu.VMEM((b_per_w,), jnp.int32),
            pltpu.VMEM((_C, _D), jnp.float32),
            pltpu.VMEM((_C, _D), jnp.float32),
            pltpu.SemaphoreType.DMA,
            pltpu.SemaphoreType.DMA,
        ],
    )
    def gather_kernel(table_hbm, idx_hbm, out_hbm, idx_v, rows0, rows1, sem0, sem1):
        wid = lax.axis_index("s") * _NC + lax.axis_index("c")
        base = wid * b_per_w

        pltpu.sync_copy(idx_hbm.at[pl.ds(base, b_per_w)], idx_v)

        def start_gather(c, rows, sem):
            pltpu.async_copy(table_hbm.at[idx_v.at[pl.ds(c * _C, _C)]], rows, sem)

        def wait_rows(rows, sem):
            pltpu.make_async_copy(
                out_hbm.at[pl.ds(base, _C), pl.ds(0, _D)], rows, sem).wait()

        def write_rows(c, rows):
            pltpu.sync_copy(
                rows, out_hbm.at[pl.ds(base + c * _C, _C), pl.ds(0, _D)])

        start_gather(0, rows0, sem0)

        @pl.loop(0, n_chunks, step=2)
        def _(t):
            start_gather(t + 1, rows1, sem1)
            wait_rows(rows0, sem0)
            write_rows(t, rows0)
            start_gather(jnp.minimum(t + 2, n_chunks - 2), rows0, sem0)
            wait_rows(rows1, sem1)
            write_rows(t + 1, rows1)

        wait_rows(rows0, sem0)

    out_padded = gather_kernel(table, idx)
    return out_padded[:, :_D].reshape(b, s, _D)
